# baseline (speedup 1.0000x reference)
import jax
import jax.numpy as jnp
from jax.experimental import pallas as pl
from jax.experimental.pallas import tpu as pltpu

_BUFF = 16
_PPF = 80
_PATCH2 = 9
_C = 128
_H = 128
_W = 128
_DS = 4
_FLS_H = 512.0
_FLS_W = 512.0
_R_MIN = 0.5
_R_MAX = 30.0
_FOV_H = 130.0

_CB = 8
_NC = _C // _CB
_PK = _PPF * _C * _PATCH2 // _PPF


def _big_body(scal_ref, fmap_ref, imapf_ref, f1b_ref, ib_ref, f2b_ref,
              f1o_ref, f2o_ref, io_ref):
    r = pl.program_id(1)
    li = scal_ref[0]

    @pl.when(r == li)
    def _():
        x = fmap_ref[0]
        f1o_ref[0] = x
        io_ref[0] = imapf_ref[0]
        a = x.reshape(_CB, _H // _DS, _DS, _W).sum(axis=2)
        b = a.reshape(_CB, _H // _DS, _W // _DS, _DS).sum(axis=3)
        f2o_ref[0] = b * (1.0 / (_DS * _DS))

    @pl.when(r != li)
    def _():
        f1o_ref[0] = f1b_ref[0]
        io_ref[0] = ib_ref[0]
        f2o_ref[0] = f2b_ref[0]


def _small_body(scal_ref, ts_ref, coords_ref, patches_ref, pb_ref, psb_ref,
                tb_ref, sfb_ref, po_ref, pso_ref, to_ref, sfo_ref):
    r = pl.program_id(0)
    li = scal_ref[0]
    fn = scal_ref[1]

    @pl.when(r == li)
    def _():
        po_ref[0] = patches_ref[0]
        xy = coords_ref[0]
        rp = xy[1:2, :] * ((_R_MAX - _R_MIN) / _FLS_H) + _R_MIN
        th = (xy[0:1, :] * (1.0 / _FLS_W) - 0.5) * (_FOV_H * jnp.pi / 180.0)
        pso_ref[0] = jnp.concatenate(
            [rp, th, jnp.zeros((1, _PPF), jnp.float32)], axis=0)
        sfo_ref[0] = jnp.full((1, _PPF), fn, dtype=jnp.int32)

    @pl.when(r != li)
    def _():
        po_ref[0] = pb_ref[0]
        pso_ref[0] = psb_ref[0]
        sfo_ref[0] = sfb_ref[0]

    @pl.when(r == 0)
    def _():
        lanes = jax.lax.broadcasted_iota(jnp.int32, (1, _BUFF), 1)
        to_ref[...] = jnp.where(lanes == li, ts_ref[0, 0], tb_ref[...])


def kernel(fmap, imap, patches, coords, time_stamp, frame_n,
           fmap1_buf, fmap2_buf, imap_buf, patches_buf,
           patch_state_buf, time_buf, source_frame_buf):
    frame_n = jnp.asarray(frame_n, jnp.int32)
    li = frame_n % _BUFF
    scal = jnp.stack([li, frame_n])

    f32 = jnp.float32
    big = pl.pallas_call(
        _big_body,
        grid_spec=pltpu.PrefetchScalarGridSpec(
            num_scalar_prefetch=1,
            grid=(_NC, _BUFF),
            in_specs=[
                pl.BlockSpec((1, _CB, _H, _W), lambda c, r, s: (0, c, 0, 0)),
                pl.BlockSpec((1, _CB, _H, _W), lambda c, r, s: (0, c, 0, 0)),
                pl.BlockSpec((1, _CB, _H, _W), lambda c, r, s: (r, c, 0, 0)),
                pl.BlockSpec((1, _CB, _H, _W), lambda c, r, s: (r, c, 0, 0)),
                pl.BlockSpec((1, _CB, _H // _DS, _W // _DS),
                             lambda c, r, s: (r, c, 0, 0)),
            ],
            out_specs=[
                pl.BlockSpec((1, _CB, _H, _W), lambda c, r, s: (r, c, 0, 0)),
                pl.BlockSpec((1, _CB, _H // _DS, _W // _DS),
                             lambda c, r, s: (r, c, 0, 0)),
                pl.BlockSpec((1, _CB, _H, _W), lambda c, r, s: (r, c, 0, 0)),
            ],
        ),
        out_shape=[
            jax.ShapeDtypeStruct((_BUFF, _C, _H, _W), f32),
            jax.ShapeDtypeStruct((_BUFF, _C, _H // _DS, _W // _DS), f32),
            jax.ShapeDtypeStruct((_BUFF, _C, _H, _W), f32),
        ],
    )
    fmap1_new, fmap2_new, imap_new = big(scal, fmap, imap,
                                         fmap1_buf, imap_buf, fmap2_buf)

    pflat = patches.reshape(1, _PPF, _C * _PATCH2)
    pbflat = patches_buf.reshape(_BUFF, _PPF, _C * _PATCH2)
    coords2 = coords[0].T.reshape(1, 2, _PPF)
    ts2 = time_stamp.reshape(1, 1)
    ps3 = jnp.swapaxes(patch_state_buf, 1, 2)
    tb2 = time_buf.reshape(1, _BUFF)
    sf3 = source_frame_buf.reshape(_BUFF, 1, _PPF)

    small = pl.pallas_call(
        _small_body,
        grid_spec=pltpu.PrefetchScalarGridSpec(
            num_scalar_prefetch=1,
            grid=(_BUFF,),
            in_specs=[
                pl.BlockSpec((1, 1), lambda r, s: (0, 0)),
                pl.BlockSpec((1, 2, _PPF), lambda r, s: (0, 0, 0)),
                pl.BlockSpec((1, _PPF, _C * _PATCH2), lambda r, s: (0, 0, 0)),
                pl.BlockSpec((1, _PPF, _C * _PATCH2), lambda r, s: (r, 0, 0)),
                pl.BlockSpec((1, 3, _PPF), lambda r, s: (r, 0, 0)),
                pl.BlockSpec((1, _BUFF), lambda r, s: (0, 0)),
                pl.BlockSpec((1, 1, _PPF), lambda r, s: (r, 0, 0)),
            ],
            out_specs=[
                pl.BlockSpec((1, _PPF, _C * _PATCH2), lambda r, s: (r, 0, 0)),
                pl.BlockSpec((1, 3, _PPF), lambda r, s: (r, 0, 0)),
                pl.BlockSpec((1, _BUFF), lambda r, s: (0, 0)),
                pl.BlockSpec((1, 1, _PPF), lambda r, s: (r, 0, 0)),
            ],
        ),
        out_shape=[
            jax.ShapeDtypeStruct((_BUFF, _PPF, _C * _PATCH2), f32),
            jax.ShapeDtypeStruct((_BUFF, 3, _PPF), f32),
            jax.ShapeDtypeStruct((1, _BUFF), f32),
            jax.ShapeDtypeStruct((_BUFF, 1, _PPF), jnp.int32),
        ],
    )
    pnew, psnew, tnew, sfnew = small(scal, ts2, coords2, pflat, pbflat,
                                     ps3, tb2, sf3)

    patches_new = pnew.reshape(_BUFF, _PPF, _C, _PATCH2)
    patch_state_new = jnp.swapaxes(psnew, 1, 2)
    time_new = tnew.reshape(_BUFF)
    source_frame_new = sfnew.reshape(_BUFF, _PPF)
    return (fmap1_new, fmap2_new, imap_new, patches_new,
            patch_state_new, time_new, source_frame_new)
